# Initial kernel scaffold; baseline (speedup 1.0000x reference)
import jax
import jax.numpy as jnp
from jax.experimental import pallas as pl


def kernel(x, router_W, W1, b1, W2, b2):
    raise NotImplementedError("write your pallas kernel here")


# speedup vs baseline: 2.2908x; 2.2908x over previous
import functools

import jax
import jax.numpy as jnp
from jax import lax
from jax.experimental import pallas as pl
from jax.experimental.pallas import tpu as pltpu
from jax.experimental.pallas import tpu_sc as plsc

T = 8192
H = 1024
I = 4096
E = 8
K = 2

TT = 512
TM = 256
TI = 512
PMAX = K * T + E * TM
NMT = PMAX // TM
NI = I // TI

NC = 2
NS = 16
NW = NC * NS
GCH = 64
TTC = 512

_SQRT_HALF = 0.7071067811865476


def _router_body(x_ref, rw_ref, e0_ref, e1_ref, w0_ref, w1_ref, stats_ref,
                 aux_ref):
    logits = jnp.dot(x_ref[...], rw_ref[...],
                     preferred_element_type=jnp.float32)
    m = jnp.max(logits, axis=1, keepdims=True)
    ex = jnp.exp(logits - m)
    probs = ex / jnp.sum(ex, axis=1, keepdims=True)

    lane = lax.broadcasted_iota(jnp.int32, (TT, E), 1)
    p0 = jnp.max(probs, axis=1, keepdims=True)
    e0 = jnp.min(jnp.where(probs == p0, lane, E), axis=1, keepdims=True)
    masked = jnp.where(lane == e0, -jnp.inf, probs)
    p1 = jnp.max(masked, axis=1, keepdims=True)
    e1 = jnp.min(jnp.where(masked == p1, lane, E), axis=1, keepdims=True)

    s = p0 + p1
    e0_ref[...] = e0
    e1_ref[...] = e1
    w0_ref[...] = p0 / s
    w1_ref[...] = p1 / s

    psum = jnp.sum(probs, axis=0, keepdims=True)
    cnt = (jnp.sum((lane == e0).astype(jnp.float32), axis=0, keepdims=True)
           + jnp.sum((lane == e1).astype(jnp.float32), axis=0, keepdims=True))
    cur = jnp.concatenate([psum, cnt], axis=0)
    step = pl.program_id(0)
    stats = jnp.where(step == 0, cur, stats_ref[...] + cur)
    stats_ref[...] = stats

    @pl.when(step == (T // TT) - 1)
    def _():
        aux_ref[...] = jnp.sum(stats[0:1, :] * stats[1:2, :], axis=1,
                               keepdims=True) * (
            float(E) / (float(T) * float(T) * float(K)))


_router = pl.pallas_call(
    _router_body,
    grid=(T // TT,),
    in_specs=[
        pl.BlockSpec((TT, H), lambda i: (i, 0)),
        pl.BlockSpec((H, E), lambda i: (0, 0)),
    ],
    out_specs=[
        pl.BlockSpec((TT, 1), lambda i: (i, 0)),
        pl.BlockSpec((TT, 1), lambda i: (i, 0)),
        pl.BlockSpec((TT, 1), lambda i: (i, 0)),
        pl.BlockSpec((TT, 1), lambda i: (i, 0)),
        pl.BlockSpec((2, E), lambda i: (0, 0)),
        pl.BlockSpec((1, 1), lambda i: (0, 0)),
    ],
    out_shape=[
        jax.ShapeDtypeStruct((T, 1), jnp.int32),
        jax.ShapeDtypeStruct((T, 1), jnp.int32),
        jax.ShapeDtypeStruct((T, 1), jnp.float32),
        jax.ShapeDtypeStruct((T, 1), jnp.float32),
        jax.ShapeDtypeStruct((2, E), jnp.float32),
        jax.ShapeDtypeStruct((1, 1), jnp.float32),
    ],
    compiler_params=pltpu.CompilerParams(
        dimension_semantics=("arbitrary",)),
)


def _dispatch_meta(e0, e1, w0, w1):
    eslot = jnp.stack([e0, e1], axis=1).reshape(-1)
    wslot = jnp.stack([w0, w1], axis=1).reshape(-1)
    perm = jnp.argsort(eslot)
    sizes = jnp.zeros((E,), jnp.int32).at[eslot].add(1)
    start = jnp.concatenate([jnp.zeros((1,), jnp.int32),
                             jnp.cumsum(sizes)[:-1]])
    padded = ((sizes + TM - 1) // TM) * TM
    pcum = jnp.cumsum(padded)
    pstart = jnp.concatenate([jnp.zeros((1,), jnp.int32), pcum[:-1]])
    j = jnp.arange(T * K, dtype=jnp.int32)
    es = eslot[perm]
    p = pstart[es] + (j - start[es])
    src_token = jnp.zeros((PMAX,), jnp.int32).at[p].set(
        (perm // K).astype(jnp.int32))
    wpad = jnp.zeros((PMAX, 1), jnp.float32).at[p, 0].set(wslot[perm])
    poss = jnp.zeros((T * K,), jnp.int32).at[perm].set(p)
    emap = jnp.minimum(
        jnp.searchsorted(pcum, jnp.arange(NMT, dtype=jnp.int32) * TM,
                         side="right"),
        E - 1).astype(jnp.int32)
    return src_token, wpad, poss, emap


@functools.lru_cache(maxsize=None)
def _sc_gather_kernel(n_src, n_out):
    mesh = plsc.VectorSubcoreMesh(core_axis_name="c", subcore_axis_name="s",
                                  num_cores=NC, num_subcores=NS)
    bpw = n_out // NW

    @functools.partial(
        pl.kernel,
        out_type=jax.ShapeDtypeStruct((n_out, H), jnp.float32),
        mesh=mesh,
        scratch_types=[
            pltpu.VMEM((GCH,), jnp.int32),
            pltpu.VMEM((GCH, H), jnp.float32),
            pltpu.SemaphoreType.DMA,
        ],
    )
    def _sc_gather(src_hbm, idx_hbm, out_hbm, idx_v, rows_v, sem):
        wid = lax.axis_index("s") * NC + lax.axis_index("c")
        base = wid * bpw
        for c in range(bpw // GCH):
            off = base + c * GCH
            pltpu.sync_copy(idx_hbm.at[pl.ds(off, GCH)], idx_v)
            pltpu.async_copy(src_hbm.at[idx_v], rows_v, sem).wait()
            pltpu.sync_copy(rows_v, out_hbm.at[pl.ds(off, GCH)])

    return _sc_gather


def _ffn_body(em_ref, xg_ref, w1_ref, b1_ref, w2_ref, b2_ref, wp_ref,
              out_ref):
    i = pl.program_id(1)
    h = jnp.dot(xg_ref[...], w1_ref[0],
                preferred_element_type=jnp.float32) + b1_ref[0]
    h = 0.5 * h * (1.0 + lax.erf(h * _SQRT_HALF))
    part = jnp.dot(h, w2_ref[0], preferred_element_type=jnp.float32)
    w = wp_ref[...]

    @pl.when(i == 0)
    def _():
        out_ref[...] = w * (part + b2_ref[0])

    @pl.when(i > 0)
    def _():
        out_ref[...] += w * part


_ffn = pl.pallas_call(
    _ffn_body,
    grid_spec=pltpu.PrefetchScalarGridSpec(
        num_scalar_prefetch=1,
        grid=(NMT, NI),
        in_specs=[
            pl.BlockSpec((TM, H), lambda m, i, em: (m, 0)),
            pl.BlockSpec((1, H, TI), lambda m, i, em: (em[m], 0, i)),
            pl.BlockSpec((1, 1, TI), lambda m, i, em: (em[m], 0, i)),
            pl.BlockSpec((1, TI, H), lambda m, i, em: (em[m], i, 0)),
            pl.BlockSpec((1, 1, H), lambda m, i, em: (em[m], 0, 0)),
            pl.BlockSpec((TM, 1), lambda m, i, em: (m, 0)),
        ],
        out_specs=pl.BlockSpec((TM, H), lambda m, i, em: (m, 0)),
    ),
    out_shape=jax.ShapeDtypeStruct((PMAX, H), jnp.float32),
    compiler_params=pltpu.CompilerParams(
        dimension_semantics=("arbitrary", "arbitrary")),
)


def _combine_body(a_ref, b_ref, out_ref):
    out_ref[...] = a_ref[...] + b_ref[...]


_combine = pl.pallas_call(
    _combine_body,
    grid=(T // TTC,),
    in_specs=[
        pl.BlockSpec((TTC, H), lambda m: (m, 0)),
        pl.BlockSpec((TTC, H), lambda m: (m, 1)),
    ],
    out_specs=pl.BlockSpec((TTC, H), lambda m: (m, 0)),
    out_shape=jax.ShapeDtypeStruct((T, H), jnp.float32),
    compiler_params=pltpu.CompilerParams(
        dimension_semantics=("arbitrary",)),
)


def kernel(x, router_W, W1, b1, W2, b2):
    B, S, h = x.shape
    flat_x = x.reshape(-1, h)
    e0, e1, w0, w1, stats, aux = _router(flat_x, router_W)
    src_token, wpad, poss, emap = _dispatch_meta(
        e0[:, 0], e1[:, 0], w0[:, 0], w1[:, 0])
    xg = _sc_gather_kernel(T, PMAX)(flat_x, src_token)
    y = _ffn(emap, xg, W1, b1.reshape(E, 1, I), W2, b2.reshape(E, 1, h), wpad)
    yg = _sc_gather_kernel(PMAX, T * K)(y, poss)
    ygr = yg.reshape(T, K * h)
    out = _combine(ygr, ygr)
    return out.reshape(B, S, h), aux.reshape(())


# speedup vs baseline: 3.0765x; 1.3430x over previous
import functools

import jax
import jax.numpy as jnp
from jax import lax
from jax.experimental import pallas as pl
from jax.experimental.pallas import tpu as pltpu
from jax.experimental.pallas import tpu_sc as plsc

T = 8192
H = 1024
I = 4096
E = 8
K = 2

TT = 512
TM = 256
TI = 512
PMAX = K * T + E * TM
NMT = PMAX // TM
NI = I // TI

NC = 2
NS = 16
NW = NC * NS
GCH = 64
TTC = 512

_SQRT_HALF = 0.7071067811865476


def _router_body(x_ref, rw_ref, e0_ref, e1_ref, w0_ref, w1_ref, stats_ref,
                 aux_ref):
    logits = jnp.dot(x_ref[...], rw_ref[...],
                     preferred_element_type=jnp.float32)
    m = jnp.max(logits, axis=1, keepdims=True)
    ex = jnp.exp(logits - m)
    probs = ex / jnp.sum(ex, axis=1, keepdims=True)

    lane = lax.broadcasted_iota(jnp.int32, (TT, E), 1)
    p0 = jnp.max(probs, axis=1, keepdims=True)
    e0 = jnp.min(jnp.where(probs == p0, lane, E), axis=1, keepdims=True)
    masked = jnp.where(lane == e0, -jnp.inf, probs)
    p1 = jnp.max(masked, axis=1, keepdims=True)
    e1 = jnp.min(jnp.where(masked == p1, lane, E), axis=1, keepdims=True)

    s = p0 + p1
    e0_ref[...] = e0
    e1_ref[...] = e1
    w0_ref[...] = p0 / s
    w1_ref[...] = p1 / s

    psum = jnp.sum(probs, axis=0, keepdims=True)
    cnt = (jnp.sum((lane == e0).astype(jnp.float32), axis=0, keepdims=True)
           + jnp.sum((lane == e1).astype(jnp.float32), axis=0, keepdims=True))
    cur = jnp.concatenate([psum, cnt], axis=0)
    step = pl.program_id(0)
    stats = jnp.where(step == 0, cur, stats_ref[...] + cur)
    stats_ref[...] = stats

    @pl.when(step == (T // TT) - 1)
    def _():
        aux_ref[...] = jnp.sum(stats[0:1, :] * stats[1:2, :], axis=1,
                               keepdims=True) * (
            float(E) / (float(T) * float(T) * float(K)))


_router = pl.pallas_call(
    _router_body,
    grid=(T // TT,),
    in_specs=[
        pl.BlockSpec((TT, H), lambda i: (i, 0)),
        pl.BlockSpec((H, E), lambda i: (0, 0)),
    ],
    out_specs=[
        pl.BlockSpec((TT, 1), lambda i: (i, 0)),
        pl.BlockSpec((TT, 1), lambda i: (i, 0)),
        pl.BlockSpec((TT, 1), lambda i: (i, 0)),
        pl.BlockSpec((TT, 1), lambda i: (i, 0)),
        pl.BlockSpec((2, E), lambda i: (0, 0)),
        pl.BlockSpec((1, 1), lambda i: (0, 0)),
    ],
    out_shape=[
        jax.ShapeDtypeStruct((T, 1), jnp.int32),
        jax.ShapeDtypeStruct((T, 1), jnp.int32),
        jax.ShapeDtypeStruct((T, 1), jnp.float32),
        jax.ShapeDtypeStruct((T, 1), jnp.float32),
        jax.ShapeDtypeStruct((2, E), jnp.float32),
        jax.ShapeDtypeStruct((1, 1), jnp.float32),
    ],
    compiler_params=pltpu.CompilerParams(
        dimension_semantics=("arbitrary",)),
)


def _dispatch_meta(e0, e1, w0, w1):
    eslot = jnp.stack([e0, e1], axis=1).reshape(-1)
    wslot = jnp.stack([w0, w1], axis=1).reshape(-1)
    perm = jnp.argsort(eslot)
    sizes = jnp.zeros((E,), jnp.int32).at[eslot].add(1)
    start = jnp.concatenate([jnp.zeros((1,), jnp.int32),
                             jnp.cumsum(sizes)[:-1]])
    padded = ((sizes + TM - 1) // TM) * TM
    pcum = jnp.cumsum(padded)
    pstart = jnp.concatenate([jnp.zeros((1,), jnp.int32), pcum[:-1]])
    j = jnp.arange(T * K, dtype=jnp.int32)
    es = eslot[perm]
    p = pstart[es] + (j - start[es])
    src_token = jnp.zeros((PMAX,), jnp.int32).at[p].set(
        (perm // K).astype(jnp.int32))
    wpad = jnp.zeros((PMAX, 1), jnp.float32).at[p, 0].set(wslot[perm])
    poss = jnp.zeros((T * K,), jnp.int32).at[perm].set(p)
    emap = jnp.minimum(
        jnp.searchsorted(pcum, jnp.arange(NMT, dtype=jnp.int32) * TM,
                         side="right"),
        E - 1).astype(jnp.int32)
    return src_token, wpad, poss, emap


@functools.lru_cache(maxsize=None)
def _sc_gather_kernel(n_src, n_out):
    mesh = plsc.VectorSubcoreMesh(core_axis_name="c", subcore_axis_name="s",
                                  num_cores=NC, num_subcores=NS)
    bpw = n_out // NW

    @functools.partial(
        pl.kernel,
        out_type=jax.ShapeDtypeStruct((n_out, H), jnp.float32),
        mesh=mesh,
        scratch_types=[
            pltpu.VMEM((GCH,), jnp.int32),
            pltpu.VMEM((GCH, H), jnp.float32),
            pltpu.SemaphoreType.DMA,
        ],
    )
    def _sc_gather(src_hbm, idx_hbm, out_hbm, idx_v, rows_v, sem):
        wid = lax.axis_index("s") * NC + lax.axis_index("c")
        base = wid * bpw
        for c in range(bpw // GCH):
            off = base + c * GCH
            pltpu.sync_copy(idx_hbm.at[pl.ds(off, GCH)], idx_v)
            pltpu.async_copy(src_hbm.at[idx_v], rows_v, sem).wait()
            pltpu.sync_copy(rows_v, out_hbm.at[pl.ds(off, GCH)])

    return _sc_gather


def _ffn_body(em_ref, xg_ref, w1_ref, b1_ref, w2_ref, b2_ref, wp_ref,
              out_ref):
    xb = xg_ref[...].astype(jnp.bfloat16)
    h = jnp.dot(xb, w1_ref[0], preferred_element_type=jnp.float32) + b1_ref[0]
    h = 0.5 * h * (1.0 + lax.erf(h * _SQRT_HALF))
    part = jnp.dot(h.astype(jnp.bfloat16), w2_ref[0],
                   preferred_element_type=jnp.float32)
    out_ref[...] = wp_ref[...] * (part + b2_ref[0])


_ffn = pl.pallas_call(
    _ffn_body,
    grid_spec=pltpu.PrefetchScalarGridSpec(
        num_scalar_prefetch=1,
        grid=(NMT,),
        in_specs=[
            pl.BlockSpec((TM, H), lambda m, em: (m, 0)),
            pl.BlockSpec((1, H, I), lambda m, em: (em[m], 0, 0)),
            pl.BlockSpec((1, 1, I), lambda m, em: (em[m], 0, 0)),
            pl.BlockSpec((1, I, H), lambda m, em: (em[m], 0, 0)),
            pl.BlockSpec((1, 1, H), lambda m, em: (em[m], 0, 0)),
            pl.BlockSpec((TM, 1), lambda m, em: (m, 0)),
        ],
        out_specs=pl.BlockSpec((TM, H), lambda m, em: (m, 0)),
    ),
    out_shape=jax.ShapeDtypeStruct((PMAX, H), jnp.float32),
    compiler_params=pltpu.CompilerParams(
        dimension_semantics=("arbitrary",)),
)


def _combine_body(a_ref, b_ref, out_ref):
    out_ref[...] = a_ref[...] + b_ref[...]


_combine = pl.pallas_call(
    _combine_body,
    grid=(T // TTC,),
    in_specs=[
        pl.BlockSpec((TTC, H), lambda m: (m, 0)),
        pl.BlockSpec((TTC, H), lambda m: (m, 1)),
    ],
    out_specs=pl.BlockSpec((TTC, H), lambda m: (m, 0)),
    out_shape=jax.ShapeDtypeStruct((T, H), jnp.float32),
    compiler_params=pltpu.CompilerParams(
        dimension_semantics=("arbitrary",)),
)


def kernel(x, router_W, W1, b1, W2, b2):
    B, S, h = x.shape
    flat_x = x.reshape(-1, h)
    e0, e1, w0, w1, stats, aux = _router(flat_x, router_W)
    src_token, wpad, poss, emap = _dispatch_meta(
        e0[:, 0], e1[:, 0], w0[:, 0], w1[:, 0])
    xg = _sc_gather_kernel(T, PMAX)(flat_x, src_token)
    y = _ffn(emap, xg, W1.astype(jnp.bfloat16), b1.reshape(E, 1, I),
             W2.astype(jnp.bfloat16), b2.reshape(E, 1, h), wpad)
    yg = _sc_gather_kernel(PMAX, T * K)(y, poss)
    ygr = yg.reshape(T, K * h)
    out = _combine(ygr, ygr)
    return out.reshape(B, S, h), aux.reshape(())


# speedup vs baseline: 3.0787x; 1.0007x over previous
import functools

import jax
import jax.numpy as jnp
from jax import lax
from jax.experimental import pallas as pl
from jax.experimental.pallas import tpu as pltpu
from jax.experimental.pallas import tpu_sc as plsc

T = 8192
H = 1024
I = 4096
E = 8
K = 2

TT = 512
TM = 256
TI = 512
PMAX = K * T + E * TM
NMT = PMAX // TM
NI = I // TI

NC = 2
NS = 16
NW = NC * NS
GCH = 64
TTC = 512

_SQRT_HALF = 0.7071067811865476


def _router_body(x_ref, rw_ref, e0_ref, e1_ref, w0_ref, w1_ref, stats_ref,
                 aux_ref):
    logits = jnp.dot(x_ref[...], rw_ref[...],
                     preferred_element_type=jnp.float32)
    m = jnp.max(logits, axis=1, keepdims=True)
    ex = jnp.exp(logits - m)
    probs = ex / jnp.sum(ex, axis=1, keepdims=True)

    lane = lax.broadcasted_iota(jnp.int32, (TT, E), 1)
    p0 = jnp.max(probs, axis=1, keepdims=True)
    e0 = jnp.min(jnp.where(probs == p0, lane, E), axis=1, keepdims=True)
    masked = jnp.where(lane == e0, -jnp.inf, probs)
    p1 = jnp.max(masked, axis=1, keepdims=True)
    e1 = jnp.min(jnp.where(masked == p1, lane, E), axis=1, keepdims=True)

    s = p0 + p1
    e0_ref[...] = e0
    e1_ref[...] = e1
    w0_ref[...] = p0 / s
    w1_ref[...] = p1 / s

    psum = jnp.sum(probs, axis=0, keepdims=True)
    cnt = (jnp.sum((lane == e0).astype(jnp.float32), axis=0, keepdims=True)
           + jnp.sum((lane == e1).astype(jnp.float32), axis=0, keepdims=True))
    cur = jnp.concatenate([psum, cnt], axis=0)
    step = pl.program_id(0)
    stats = jnp.where(step == 0, cur, stats_ref[...] + cur)
    stats_ref[...] = stats

    @pl.when(step == (T // TT) - 1)
    def _():
        aux_ref[...] = jnp.sum(stats[0:1, :] * stats[1:2, :], axis=1,
                               keepdims=True) * (
            float(E) / (float(T) * float(T) * float(K)))


_router = pl.pallas_call(
    _router_body,
    grid=(T // TT,),
    in_specs=[
        pl.BlockSpec((TT, H), lambda i: (i, 0)),
        pl.BlockSpec((H, E), lambda i: (0, 0)),
    ],
    out_specs=[
        pl.BlockSpec((TT, 1), lambda i: (i, 0)),
        pl.BlockSpec((TT, 1), lambda i: (i, 0)),
        pl.BlockSpec((TT, 1), lambda i: (i, 0)),
        pl.BlockSpec((TT, 1), lambda i: (i, 0)),
        pl.BlockSpec((2, E), lambda i: (0, 0)),
        pl.BlockSpec((1, 1), lambda i: (0, 0)),
    ],
    out_shape=[
        jax.ShapeDtypeStruct((T, 1), jnp.int32),
        jax.ShapeDtypeStruct((T, 1), jnp.int32),
        jax.ShapeDtypeStruct((T, 1), jnp.float32),
        jax.ShapeDtypeStruct((T, 1), jnp.float32),
        jax.ShapeDtypeStruct((2, E), jnp.float32),
        jax.ShapeDtypeStruct((1, 1), jnp.float32),
    ],
    compiler_params=pltpu.CompilerParams(
        dimension_semantics=("arbitrary",)),
)


def _dispatch_meta(e0, e1, w0, w1):
    eslot = jnp.stack([e0, e1], axis=1).reshape(-1)
    wslot = jnp.stack([w0, w1], axis=1).reshape(-1)
    perm = jnp.argsort(eslot)
    sizes = jnp.zeros((E,), jnp.int32).at[eslot].add(1)
    start = jnp.concatenate([jnp.zeros((1,), jnp.int32),
                             jnp.cumsum(sizes)[:-1]])
    padded = ((sizes + TM - 1) // TM) * TM
    pcum = jnp.cumsum(padded)
    pstart = jnp.concatenate([jnp.zeros((1,), jnp.int32), pcum[:-1]])
    j = jnp.arange(T * K, dtype=jnp.int32)
    es = eslot[perm]
    p = pstart[es] + (j - start[es])
    src_token = jnp.zeros((PMAX,), jnp.int32).at[p].set(
        (perm // K).astype(jnp.int32))
    wpad = jnp.zeros((PMAX, 1), jnp.float32).at[p, 0].set(wslot[perm])
    poss = jnp.zeros((T * K,), jnp.int32).at[perm].set(p)
    emap = jnp.minimum(
        jnp.searchsorted(pcum, jnp.arange(NMT, dtype=jnp.int32) * TM,
                         side="right"),
        E - 1).astype(jnp.int32)
    return src_token, wpad, poss, emap


@functools.lru_cache(maxsize=None)
def _sc_gather_kernel(n_src, n_out, gch):
    mesh = plsc.VectorSubcoreMesh(core_axis_name="c", subcore_axis_name="s",
                                  num_cores=NC, num_subcores=NS)
    bpw = n_out // NW
    nch = bpw // gch

    @functools.partial(
        pl.kernel,
        out_type=jax.ShapeDtypeStruct((n_out, H), jnp.float32),
        mesh=mesh,
        scratch_types=[
            pltpu.VMEM((bpw,), jnp.int32),
            pltpu.VMEM((gch, H), jnp.float32),
            pltpu.VMEM((gch, H), jnp.float32),
            pltpu.SemaphoreType.DMA,
            pltpu.SemaphoreType.DMA,
        ],
    )
    def _sc_gather(src_hbm, idx_hbm, out_hbm, idx_v, rows_v0, rows_v1, sem0,
                   sem1):
        wid = lax.axis_index("s") * NC + lax.axis_index("c")
        base = wid * bpw
        pltpu.sync_copy(idx_hbm.at[pl.ds(base, bpw)], idx_v)
        bufs = (rows_v0, rows_v1)
        sems = (sem0, sem1)
        cps = [None, None]
        cps[0] = pltpu.async_copy(
            src_hbm.at[idx_v.at[pl.ds(0, gch)]], bufs[0], sems[0])
        for c in range(nch):
            b = c & 1
            if c + 1 < nch:
                nb = (c + 1) & 1
                cps[nb] = pltpu.async_copy(
                    src_hbm.at[idx_v.at[pl.ds((c + 1) * gch, gch)]],
                    bufs[nb], sems[nb])
            cps[b].wait()
            pltpu.sync_copy(bufs[b], out_hbm.at[pl.ds(base + c * gch, gch)])

    return _sc_gather


def _ffn_body(em_ref, xg_ref, w1_ref, b1_ref, w2_ref, b2_ref, wp_ref,
              out_ref):
    xb = xg_ref[...].astype(jnp.bfloat16)
    h = jnp.dot(xb, w1_ref[0], preferred_element_type=jnp.float32) + b1_ref[0]
    h = 0.5 * h * (1.0 + lax.erf(h * _SQRT_HALF))
    part = jnp.dot(h.astype(jnp.bfloat16), w2_ref[0],
                   preferred_element_type=jnp.float32)
    out_ref[...] = wp_ref[...] * (part + b2_ref[0])


_ffn = pl.pallas_call(
    _ffn_body,
    grid_spec=pltpu.PrefetchScalarGridSpec(
        num_scalar_prefetch=1,
        grid=(NMT,),
        in_specs=[
            pl.BlockSpec((TM, H), lambda m, em: (m, 0)),
            pl.BlockSpec((1, H, I), lambda m, em: (em[m], 0, 0)),
            pl.BlockSpec((1, 1, I), lambda m, em: (em[m], 0, 0)),
            pl.BlockSpec((1, I, H), lambda m, em: (em[m], 0, 0)),
            pl.BlockSpec((1, 1, H), lambda m, em: (em[m], 0, 0)),
            pl.BlockSpec((TM, 1), lambda m, em: (m, 0)),
        ],
        out_specs=pl.BlockSpec((TM, H), lambda m, em: (m, 0)),
    ),
    out_shape=jax.ShapeDtypeStruct((PMAX, H), jnp.float32),
    compiler_params=pltpu.CompilerParams(
        dimension_semantics=("arbitrary",)),
)


def _combine_body(a_ref, b_ref, out_ref):
    out_ref[...] = a_ref[...] + b_ref[...]


_combine = pl.pallas_call(
    _combine_body,
    grid=(T // TTC,),
    in_specs=[
        pl.BlockSpec((TTC, H), lambda m: (m, 0)),
        pl.BlockSpec((TTC, H), lambda m: (m, 1)),
    ],
    out_specs=pl.BlockSpec((TTC, H), lambda m: (m, 0)),
    out_shape=jax.ShapeDtypeStruct((T, H), jnp.float32),
    compiler_params=pltpu.CompilerParams(
        dimension_semantics=("arbitrary",)),
)


def kernel(x, router_W, W1, b1, W2, b2):
    B, S, h = x.shape
    flat_x = x.reshape(-1, h)
    e0, e1, w0, w1, stats, aux = _router(flat_x, router_W)
    src_token, wpad, poss, emap = _dispatch_meta(
        e0[:, 0], e1[:, 0], w0[:, 0], w1[:, 0])
    xg = _sc_gather_kernel(T, PMAX, 48)(flat_x, src_token)
    y = _ffn(emap, xg, W1.astype(jnp.bfloat16), b1.reshape(E, 1, I),
             W2.astype(jnp.bfloat16), b2.reshape(E, 1, h), wpad)
    yg = _sc_gather_kernel(PMAX, T * K, 32)(y, poss)
    ygr = yg.reshape(T, K * h)
    out = _combine(ygr, ygr)
    return out.reshape(B, S, h), aux.reshape(())


# speedup vs baseline: 3.5843x; 1.1642x over previous
import functools

import jax
import jax.numpy as jnp
from jax import lax
from jax.experimental import pallas as pl
from jax.experimental.pallas import tpu as pltpu
from jax.experimental.pallas import tpu_sc as plsc

T = 8192
H = 1024
I = 4096
E = 8
K = 2

TT = 512
TM = 256
TI = 512
PMAX = K * T + E * TM
NMT = PMAX // TM
NI = I // TI

NC = 2
NS = 16
NW = NC * NS
GCH = 64
TTC = 512

_SQRT_HALF = 0.7071067811865476


def _router_body(x_ref, rw_ref, e0_ref, e1_ref, w0_ref, w1_ref, stats_ref,
                 aux_ref):
    logits = jnp.dot(x_ref[...], rw_ref[...],
                     preferred_element_type=jnp.float32)
    m = jnp.max(logits, axis=1, keepdims=True)
    ex = jnp.exp(logits - m)
    probs = ex / jnp.sum(ex, axis=1, keepdims=True)

    lane = lax.broadcasted_iota(jnp.int32, (TT, E), 1)
    p0 = jnp.max(probs, axis=1, keepdims=True)
    e0 = jnp.min(jnp.where(probs == p0, lane, E), axis=1, keepdims=True)
    masked = jnp.where(lane == e0, -jnp.inf, probs)
    p1 = jnp.max(masked, axis=1, keepdims=True)
    e1 = jnp.min(jnp.where(masked == p1, lane, E), axis=1, keepdims=True)

    s = p0 + p1
    e0_ref[...] = e0
    e1_ref[...] = e1
    w0_ref[...] = p0 / s
    w1_ref[...] = p1 / s

    psum = jnp.sum(probs, axis=0, keepdims=True)
    cnt = (jnp.sum((lane == e0).astype(jnp.float32), axis=0, keepdims=True)
           + jnp.sum((lane == e1).astype(jnp.float32), axis=0, keepdims=True))
    cur = jnp.concatenate([psum, cnt], axis=0)
    step = pl.program_id(0)
    stats = jnp.where(step == 0, cur, stats_ref[...] + cur)
    stats_ref[...] = stats

    @pl.when(step == (T // TT) - 1)
    def _():
        aux_ref[...] = jnp.sum(stats[0:1, :] * stats[1:2, :], axis=1,
                               keepdims=True) * (
            float(E) / (float(T) * float(T) * float(K)))


_router = pl.pallas_call(
    _router_body,
    grid=(T // TT,),
    in_specs=[
        pl.BlockSpec((TT, H), lambda i: (i, 0)),
        pl.BlockSpec((H, E), lambda i: (0, 0)),
    ],
    out_specs=[
        pl.BlockSpec((TT, 1), lambda i: (i, 0)),
        pl.BlockSpec((TT, 1), lambda i: (i, 0)),
        pl.BlockSpec((TT, 1), lambda i: (i, 0)),
        pl.BlockSpec((TT, 1), lambda i: (i, 0)),
        pl.BlockSpec((2, E), lambda i: (0, 0)),
        pl.BlockSpec((1, 1), lambda i: (0, 0)),
    ],
    out_shape=[
        jax.ShapeDtypeStruct((T, 1), jnp.int32),
        jax.ShapeDtypeStruct((T, 1), jnp.int32),
        jax.ShapeDtypeStruct((T, 1), jnp.float32),
        jax.ShapeDtypeStruct((T, 1), jnp.float32),
        jax.ShapeDtypeStruct((2, E), jnp.float32),
        jax.ShapeDtypeStruct((1, 1), jnp.float32),
    ],
    compiler_params=pltpu.CompilerParams(
        dimension_semantics=("arbitrary",)),
)


def _dispatch_meta(e0, e1, w0, w1):
    eslot = jnp.stack([e0, e1], axis=1).reshape(-1)
    wslot = jnp.stack([w0, w1], axis=1).reshape(-1)
    onehot = (eslot[:, None] == jnp.arange(E, dtype=jnp.int32)[None, :]
              ).astype(jnp.int32)
    C = jnp.cumsum(onehot, axis=0)
    sizes = C[-1]
    start = jnp.concatenate([jnp.zeros((1,), jnp.int32),
                             jnp.cumsum(sizes)[:-1]])
    padded = ((sizes + TM - 1) // TM) * TM
    pcum = jnp.cumsum(padded)
    pstart = jnp.concatenate([jnp.zeros((1,), jnp.int32), pcum[:-1]])
    rank = jnp.sum(C * onehot, axis=1) - 1
    poss = pstart[eslot] + rank
    perm = jnp.argsort(eslot)
    pidx = jnp.arange(PMAX, dtype=jnp.int32)
    row_e = jnp.minimum(jnp.searchsorted(pcum, pidx, side="right"),
                        E - 1).astype(jnp.int32)
    r = pidx - pstart[row_e]
    valid = r < sizes[row_e]
    j = jnp.clip(start[row_e] + jnp.minimum(r, jnp.maximum(
        sizes[row_e] - 1, 0)), 0, T * K - 1)
    src = perm[j]
    src_token = jnp.where(valid, src // K, 0).astype(jnp.int32)
    wpad = jnp.where(valid, wslot[src], 0.0)[:, None]
    emap = jnp.minimum(
        jnp.searchsorted(pcum, jnp.arange(NMT, dtype=jnp.int32) * TM,
                         side="right"),
        E - 1).astype(jnp.int32)
    poss2 = jnp.concatenate([poss[0::K], poss[1::K]])
    return src_token, wpad, poss2, emap


@functools.lru_cache(maxsize=None)
def _sc_gather_kernel(n_src, n_out, gch):
    mesh = plsc.VectorSubcoreMesh(core_axis_name="c", subcore_axis_name="s",
                                  num_cores=NC, num_subcores=NS)
    bpw = n_out // NW
    nch = bpw // gch

    @functools.partial(
        pl.kernel,
        out_type=jax.ShapeDtypeStruct((n_out, H), jnp.float32),
        mesh=mesh,
        scratch_types=[
            pltpu.VMEM((bpw,), jnp.int32),
            pltpu.VMEM((gch, H), jnp.float32),
            pltpu.VMEM((gch, H), jnp.float32),
            pltpu.SemaphoreType.DMA,
            pltpu.SemaphoreType.DMA,
        ],
    )
    def _sc_gather(src_hbm, idx_hbm, out_hbm, idx_v, rows_v0, rows_v1, sem0,
                   sem1):
        wid = lax.axis_index("s") * NC + lax.axis_index("c")
        base = wid * bpw
        pltpu.sync_copy(idx_hbm.at[pl.ds(base, bpw)], idx_v)
        bufs = (rows_v0, rows_v1)
        sems = (sem0, sem1)
        cps = [None, None]
        cps[0] = pltpu.async_copy(
            src_hbm.at[idx_v.at[pl.ds(0, gch)]], bufs[0], sems[0])
        for c in range(nch):
            b = c & 1
            if c + 1 < nch:
                nb = (c + 1) & 1
                cps[nb] = pltpu.async_copy(
                    src_hbm.at[idx_v.at[pl.ds((c + 1) * gch, gch)]],
                    bufs[nb], sems[nb])
            cps[b].wait()
            pltpu.sync_copy(bufs[b], out_hbm.at[pl.ds(base + c * gch, gch)])

    return _sc_gather


def _ffn_body(em_ref, xg_ref, w1_ref, b1_ref, w2_ref, b2_ref, wp_ref,
              out_ref):
    xb = xg_ref[...].astype(jnp.bfloat16)
    h = jnp.dot(xb, w1_ref[0], preferred_element_type=jnp.float32) + b1_ref[0]
    h = 0.5 * h * (1.0 + lax.erf(h * _SQRT_HALF))
    part = jnp.dot(h.astype(jnp.bfloat16), w2_ref[0],
                   preferred_element_type=jnp.float32)
    out_ref[...] = wp_ref[...] * (part + b2_ref[0])


_ffn = pl.pallas_call(
    _ffn_body,
    grid_spec=pltpu.PrefetchScalarGridSpec(
        num_scalar_prefetch=1,
        grid=(NMT,),
        in_specs=[
            pl.BlockSpec((TM, H), lambda m, em: (m, 0)),
            pl.BlockSpec((1, H, I), lambda m, em: (em[m], 0, 0)),
            pl.BlockSpec((1, 1, I), lambda m, em: (em[m], 0, 0)),
            pl.BlockSpec((1, I, H), lambda m, em: (em[m], 0, 0)),
            pl.BlockSpec((1, 1, H), lambda m, em: (em[m], 0, 0)),
            pl.BlockSpec((TM, 1), lambda m, em: (m, 0)),
        ],
        out_specs=pl.BlockSpec((TM, H), lambda m, em: (m, 0)),
    ),
    out_shape=jax.ShapeDtypeStruct((PMAX, H), jnp.float32),
    compiler_params=pltpu.CompilerParams(
        dimension_semantics=("arbitrary",)),
)


def _combine_body(a_ref, b_ref, out_ref):
    out_ref[...] = a_ref[...] + b_ref[...]


_combine = pl.pallas_call(
    _combine_body,
    grid=(T // TTC,),
    in_specs=[
        pl.BlockSpec((TTC, H), lambda m: (m, 0)),
        pl.BlockSpec((TTC, H), lambda m: (m + T // TTC, 0)),
    ],
    out_specs=pl.BlockSpec((TTC, H), lambda m: (m, 0)),
    out_shape=jax.ShapeDtypeStruct((T, H), jnp.float32),
    compiler_params=pltpu.CompilerParams(
        dimension_semantics=("arbitrary",)),
)


def _cast_body(w_ref, o_ref):
    o_ref[...] = w_ref[...].astype(jnp.bfloat16)


_w1cast = pl.pallas_call(
    _cast_body,
    grid=(E,),
    in_specs=[pl.BlockSpec((1, H, I), lambda e: (e, 0, 0))],
    out_specs=pl.BlockSpec((1, H, I), lambda e: (e, 0, 0)),
    out_shape=jax.ShapeDtypeStruct((E, H, I), jnp.bfloat16),
    compiler_params=pltpu.CompilerParams(
        dimension_semantics=("arbitrary",)),
)


def kernel(x, router_W, W1, b1, W2, b2):
    B, S, h = x.shape
    flat_x = x.reshape(-1, h)
    e0, e1, w0, w1, stats, aux = _router(flat_x, router_W)
    src_token, wpad, poss, emap = _dispatch_meta(
        e0[:, 0], e1[:, 0], w0[:, 0], w1[:, 0])
    xg = _sc_gather_kernel(T, PMAX, 48)(flat_x, src_token)
    y = _ffn(emap, xg, _w1cast(W1), b1.reshape(E, 1, I),
             W2.astype(jnp.bfloat16), b2.reshape(E, 1, h), wpad)
    yg = _sc_gather_kernel(PMAX, T * K, 32)(y, poss)
    out = _combine(yg, yg)
    return out.reshape(B, S, h), aux.reshape(())
